# Initial kernel scaffold; baseline (speedup 1.0000x reference)
import functools

import jax
import jax.numpy as jnp
from jax import lax
from jax.experimental import pallas as pl
from jax.experimental.pallas import tpu as pltpu
from jax.experimental.pallas import tpu_sc as plsc

B = 16384
C = 20
D = 64
K = 200
KPAD = 256

NW = 32
RPW = B // NW
SEG = 4
ROWS = SEG * C
NCHUNK = RPW // SEG
NBUF = 4


def _sc_kernel(rw_hbm, tw_hbm, ow_hbm, emb_v_hbm, emb_u_hbm,
               hsum_hbm, tgt_hbm, out_o_hbm,
               idx_v, tidx_v, owidx_v, gbuf, tbuf, hsum_v,
               sem0, sem1, sem2, sem3, sem_t):
    sems = [sem0, sem1, sem2, sem3]
    cid = lax.axis_index("c")
    sid = lax.axis_index("s")
    wid = cid * 16 + sid
    base_row = wid * RPW

    pltpu.sync_copy(rw_hbm.at[pl.ds(wid * NCHUNK, NCHUNK)], idx_v)
    pltpu.sync_copy(tw_hbm.at[pl.ds(wid * 4, 4)], tidx_v)

    for j in range(4):
        pltpu.async_copy(emb_v_hbm.at[tidx_v.at[j]], tbuf, sem_t).wait()
        pltpu.sync_copy(tbuf, tgt_hbm.at[pl.ds(base_row + j * 128, 128)])

    @pl.when(wid < 2)
    def _():
        pltpu.sync_copy(ow_hbm.at[wid], owidx_v)
        pltpu.async_copy(emb_u_hbm.at[owidx_v], tbuf, sem_t).wait()
        pltpu.sync_copy(tbuf, out_o_hbm.at[pl.ds(wid * 128, 128)])

    def _issue(chunk, b):
        pltpu.async_copy(emb_u_hbm.at[idx_v.at[chunk]], gbuf.at[b], sems[b])

    for b in range(NBUF - 1):
        _issue(b, b)

    @pl.loop(0, NCHUNK, step=NBUF)
    def _(i):
        for b in range(NBUF):
            chunk = i + b
            pltpu.make_async_copy(
                emb_u_hbm.at[idx_v.at[chunk]], gbuf.at[b], sems[b]).wait()
            nxt = chunk + NBUF - 1
            nb = (b + NBUF - 1) % NBUF

            @pl.when(nxt < NCHUNK)
            def _():
                _issue(nxt, nb)

            for s in range(SEG):
                r0 = s * C
                for dch in range(4):
                    dsl = pl.ds(dch * 16, 16)
                    acc = gbuf[b, r0, dsl]
                    for cc in range(1, C):
                        acc = acc + gbuf[b, r0 + cc, dsl]
                    hsum_v[chunk * SEG + s, dsl] = acc

    pltpu.sync_copy(hsum_v, hsum_hbm.at[pl.ds(base_row, RPW)])


_sc_call = functools.partial(
    pl.kernel,
    out_type=(
        jax.ShapeDtypeStruct((B, D), jnp.float32),
        jax.ShapeDtypeStruct((B, D), jnp.float32),
        jax.ShapeDtypeStruct((KPAD, D), jnp.float32),
    ),
    mesh=plsc.VectorSubcoreMesh(core_axis_name="c", subcore_axis_name="s"),
    scratch_types=[
        pltpu.VMEM((NCHUNK, ROWS), jnp.int32),
        pltpu.VMEM((4, 128), jnp.int32),
        pltpu.VMEM((128,), jnp.int32),
        pltpu.VMEM((NBUF, ROWS, D), jnp.float32),
        pltpu.VMEM((128, D), jnp.float32),
        pltpu.VMEM((RPW, D), jnp.float32),
        pltpu.SemaphoreType.DMA,
        pltpu.SemaphoreType.DMA,
        pltpu.SemaphoreType.DMA,
        pltpu.SemaphoreType.DMA,
        pltpu.SemaphoreType.DMA,
    ],
)(_sc_kernel)


BT = 2048


def _tc_kernel(t_ref, o_ref, hs_ref, out_ref):
    t = t_ref[...]
    o = o_ref[...]
    logits = lax.dot_general(t, o, (((1,), (1,)), ((), ())),
                             preferred_element_type=jnp.float32)
    kk = lax.broadcasted_iota(jnp.int32, (BT, KPAD), 1)
    e = jnp.where(kk < K, jnp.exp(logits), 0.0)
    right = jnp.log(jnp.sum(e, axis=1, keepdims=True))
    hdot = jnp.sum(t * hs_ref[...], axis=1, keepdims=True) * (1.0 / C)
    out_ref[...] = right + hdot


_tc_call = pl.pallas_call(
    _tc_kernel,
    grid=(B // BT,),
    in_specs=[
        pl.BlockSpec((BT, D), lambda i: (i, 0)),
        pl.BlockSpec((KPAD, D), lambda i: (0, 0)),
        pl.BlockSpec((BT, D), lambda i: (i, 0)),
    ],
    out_specs=pl.BlockSpec((BT, 1), lambda i: (i, 0)),
    out_shape=jax.ShapeDtypeStruct((B, 1), jnp.float32),
)


def kernel(round_words, target_words, outer_words, emb_v, emb_u):
    rw = round_words.astype(jnp.int32).reshape(B * C // ROWS, ROWS)
    tw = target_words.astype(jnp.int32).reshape(B // 128, 128)
    ow = jnp.concatenate(
        [outer_words.astype(jnp.int32),
         jnp.zeros((KPAD - K,), jnp.int32)]).reshape(2, 128)
    hsum, tgt, out_o = _sc_call(rw, tw, ow, emb_v, emb_u)
    return _tc_call(tgt, out_o, hsum)


# speedup vs baseline: 39.7578x; 39.7578x over previous
import functools

import jax
import jax.numpy as jnp
from jax import lax
from jax.experimental import pallas as pl
from jax.experimental.pallas import tpu as pltpu
from jax.experimental.pallas import tpu_sc as plsc

B = 16384
C = 20
D = 64
K = 200
KPAD = 256

NW = 32
RPW = B // NW
SEG = 4
ROWS = SEG * C
NCHUNK = RPW // SEG
NBUF = 4


def _sc_kernel(rw_hbm, tw_hbm, ow_hbm, emb_v_hbm, emb_u_hbm,
               hsum_hbm, tgt_hbm, out_o_hbm,
               idx_v, tidx_v, owidx_v, gbuf, tbuf, hsum_v,
               sem0, sem1, sem2, sem3, sem_t):
    sems = [sem0, sem1, sem2, sem3]
    cid = lax.axis_index("c")
    sid = lax.axis_index("s")
    wid = cid * 16 + sid
    base_row = wid * RPW

    pltpu.sync_copy(rw_hbm.at[pl.ds(wid * NCHUNK, NCHUNK)], idx_v)
    pltpu.sync_copy(tw_hbm.at[pl.ds(wid * 4, 4)], tidx_v)

    for j in range(4):
        pltpu.async_copy(emb_v_hbm.at[tidx_v.at[j]], tbuf, sem_t).wait()
        pltpu.sync_copy(tbuf, tgt_hbm.at[pl.ds(base_row + j * 128, 128)])

    @pl.when(wid < 2)
    def _():
        pltpu.sync_copy(ow_hbm.at[wid], owidx_v)
        pltpu.async_copy(emb_u_hbm.at[owidx_v], tbuf, sem_t).wait()
        pltpu.sync_copy(tbuf, out_o_hbm.at[pl.ds(wid * 128, 128)])

    def _issue(chunk, b):
        pltpu.async_copy(emb_u_hbm.at[idx_v.at[chunk]], gbuf.at[b], sems[b])

    for b in range(NBUF - 1):
        _issue(b, b)

    @pl.loop(0, NCHUNK, step=NBUF)
    def _(i):
        for b in range(NBUF):
            chunk = i + b
            pltpu.make_async_copy(
                emb_u_hbm.at[idx_v.at[chunk]], gbuf.at[b], sems[b]).wait()
            nxt = chunk + NBUF - 1
            nb = (b + NBUF - 1) % NBUF

            @pl.when(nxt < NCHUNK)
            def _():
                _issue(nxt, nb)

            for s in range(SEG):
                r0 = s * C
                for dch in range(4):
                    dsl = pl.ds(dch * 16, 16)
                    acc = gbuf[b, r0, dsl]
                    for cc in range(1, C):
                        acc = acc + gbuf[b, r0 + cc, dsl]
                    hsum_v[chunk * SEG + s, dsl] = acc

    pltpu.sync_copy(hsum_v, hsum_hbm.at[pl.ds(base_row, RPW)])


_sc_call = functools.partial(
    pl.kernel,
    out_type=(
        jax.ShapeDtypeStruct((B, D), jnp.float32),
        jax.ShapeDtypeStruct((B, D), jnp.float32),
        jax.ShapeDtypeStruct((KPAD, D), jnp.float32),
    ),
    mesh=plsc.VectorSubcoreMesh(core_axis_name="c", subcore_axis_name="s"),
    compiler_params=pltpu.CompilerParams(use_tc_tiling_on_sc=False),
    scratch_types=[
        pltpu.VMEM((NCHUNK, ROWS), jnp.int32),
        pltpu.VMEM((4, 128), jnp.int32),
        pltpu.VMEM((128,), jnp.int32),
        pltpu.VMEM((NBUF, ROWS, D), jnp.float32),
        pltpu.VMEM((128, D), jnp.float32),
        pltpu.VMEM((RPW, D), jnp.float32),
        pltpu.SemaphoreType.DMA,
        pltpu.SemaphoreType.DMA,
        pltpu.SemaphoreType.DMA,
        pltpu.SemaphoreType.DMA,
        pltpu.SemaphoreType.DMA,
    ],
)(_sc_kernel)


BT = 2048


def _tc_kernel(t_ref, o_ref, hs_ref, out_ref):
    t = t_ref[...]
    o = o_ref[...]
    logits = lax.dot_general(t, o, (((1,), (1,)), ((), ())),
                             preferred_element_type=jnp.float32)
    kk = lax.broadcasted_iota(jnp.int32, (BT, KPAD), 1)
    e = jnp.where(kk < K, jnp.exp(logits), 0.0)
    right = jnp.log(jnp.sum(e, axis=1, keepdims=True))
    hdot = jnp.sum(t * hs_ref[...], axis=1, keepdims=True) * (1.0 / C)
    out_ref[...] = right + hdot


_tc_call = pl.pallas_call(
    _tc_kernel,
    grid=(B // BT,),
    in_specs=[
        pl.BlockSpec((BT, D), lambda i: (i, 0)),
        pl.BlockSpec((KPAD, D), lambda i: (0, 0)),
        pl.BlockSpec((BT, D), lambda i: (i, 0)),
    ],
    out_specs=pl.BlockSpec((BT, 1), lambda i: (i, 0)),
    out_shape=jax.ShapeDtypeStruct((B, 1), jnp.float32),
)


def kernel(round_words, target_words, outer_words, emb_v, emb_u):
    rw = round_words.astype(jnp.int32).reshape(B * C // ROWS, ROWS)
    tw = target_words.astype(jnp.int32).reshape(B // 128, 128)
    ow = jnp.concatenate(
        [outer_words.astype(jnp.int32),
         jnp.zeros((KPAD - K,), jnp.int32)]).reshape(2, 128)
    hsum, tgt, out_o = _sc_call(rw, tw, ow, emb_v, emb_u)
    return _tc_call(tgt, out_o, hsum)
